# baseline (speedup 1.0000x reference)
import numpy as np

import jax
import jax.numpy as jnp
from jax import lax
from jax.experimental import pallas as pl
from jax.experimental.pallas import tpu as pltpu
from jax.experimental.pallas import tpu_sc as plsc

B = 16384
H = 200
E = 64
V = 100000
NW = 32
RW = B // NW
C = 16
NCHUNK = RW // C
NIDX = C * H // 128
HC = C // 2
FLAT = HC * H

_G0 = [(k, 0, 128, 128 * k) for k in range(12)] + [(12, 0, 64, 1536)]
_G1 = [(12, 64, 64, 0)] + [(13 + k, 0, 128, 64 + 128 * k) for k in range(12)]
_GATHERS = (_G0, _G1)


def _sc_body(table_hbm, qw_hbm, out_hbm,
             idx0, idx1, embA, embB, out_v, semA, semB):
    wid = lax.axis_index("s") * 2 + lax.axis_index("c")
    base = wid * RW
    idx_b = (idx0, idx1)
    emb_b = (embA, embB)
    sem_b = (semA, semB)

    def stage_idx(ci, p):
        pltpu.sync_copy(qw_hbm.at[pl.ds((base + ci * C) * H // 128, NIDX)],
                        idx_b[p])

    def fire(half, p, eb):
        for row, coff, n, doff in _GATHERS[half]:
            src = (idx_b[p].at[row] if n == 128
                   else idx_b[p].at[row, pl.ds(coff, n)])
            pltpu.async_copy(table_hbm.at[src],
                             emb_b[eb].at[pl.ds(doff, n)], sem_b[eb])

    def drain(half, p, eb):
        for row, coff, n, doff in _GATHERS[half]:
            src = (idx_b[p].at[row] if n == 128
                   else idx_b[p].at[row, pl.ds(coff, n)])
            pltpu.make_async_copy(table_hbm.at[src],
                                  emb_b[eb].at[pl.ds(doff, n)],
                                  sem_b[eb]).wait()

    def accum(half, eb):
        emb_v = emb_b[eb]
        for q in range(HC):
            r = half * HC + q

            def j_body(j, accs):
                e0, o0, e1, o1 = accs
                off = q * H + 4 * j

                def quad(g):
                    x = [plsc.bitcast(emb_v[off + i, pl.ds(16 * g, 16)],
                                      jnp.bfloat16) for i in range(4)]
                    return (x[0] + x[1]) + (x[2] + x[3])

                a, bb = plsc.unpack(quad(0), format=plsc.PackFormat.INTERLEAVED)
                e0, o0 = e0 + a, o0 + bb
                a, bb = plsc.unpack(quad(1), format=plsc.PackFormat.INTERLEAVED)
                e1, o1 = e1 + a, o1 + bb
                return (e0, o0, e1, o1)

            z = jnp.zeros((16,), jnp.float32)
            e0, o0, e1, o1 = lax.fori_loop(0, H // 4, j_body, (z, z, z, z),
                                           unroll=5)
            cb = 64 * (r % 2)
            out_v[r // 2, pl.ds(cb, 16)] = e0
            out_v[r // 2, pl.ds(cb + 16, 16)] = o0
            out_v[r // 2, pl.ds(cb + 32, 16)] = e1
            out_v[r // 2, pl.ds(cb + 48, 16)] = o1

    def flush_out(ci):
        pltpu.sync_copy(out_v, out_hbm.at[pl.ds((base + ci * C) // 2, HC)])

    stage_idx(0, 0)
    fire(0, 0, 0)

    def chunk_body(ci, carry):
        def do(p):
            fire(1, p, 1)
            stage_idx(ci + 1, 1 - p)
            drain(0, p, 0)
            accum(0, 0)
            fire(0, 1 - p, 0)
            drain(1, p, 1)
            accum(1, 1)
            flush_out(ci)
        lax.cond(lax.rem(ci, 2) == 0, lambda: do(0), lambda: do(1))
        return carry

    lax.fori_loop(0, NCHUNK - 1, chunk_body, 0)
    pl_last = (NCHUNK - 1) % 2
    fire(1, pl_last, 1)
    drain(0, pl_last, 0)
    accum(0, 0)
    drain(1, pl_last, 1)
    accum(1, 1)
    flush_out(NCHUNK - 1)


@jax.jit
def _sc_sums(table_i32, qw128):
    mesh = plsc.VectorSubcoreMesh(core_axis_name="c", subcore_axis_name="s")
    f = pl.kernel(
        _sc_body,
        mesh=mesh,
        compiler_params=pltpu.CompilerParams(
            use_tc_tiling_on_sc=False, needs_layout_passes=False),
        out_type=jax.ShapeDtypeStruct((B // 2, 2 * E), jnp.float32),
        scratch_types=[
            pltpu.VMEM((NIDX, 128), jnp.int32),
            pltpu.VMEM((NIDX, 128), jnp.int32),
            pltpu.VMEM((FLAT, E // 2), jnp.int32),
            pltpu.VMEM((FLAT, E // 2), jnp.int32),
            pltpu.VMEM((HC, 2 * E), jnp.float32),
            pltpu.SemaphoreType.DMA,
            pltpu.SemaphoreType.DMA,
        ],
    )
    return f(table_i32, qw128)


def _proj_body(x_ref, w_ref, b_ref, o_ref):
    o_ref[...] = jnp.tanh(
        jnp.dot(x_ref[...], w_ref[...], preferred_element_type=jnp.float32)
        + b_ref[...])


@jax.jit
def _proj(sums128, w2, b2):
    blk = 2048
    return pl.pallas_call(
        _proj_body,
        grid=(B // 2 // blk,),
        in_specs=[
            pl.BlockSpec((blk, 2 * E), lambda i: (i, 0)),
            pl.BlockSpec((2 * E, 2 * E), lambda i: (0, 0)),
            pl.BlockSpec((1, 2 * E), lambda i: (0, 0)),
        ],
        out_specs=pl.BlockSpec((blk, 2 * E), lambda i: (i, 0)),
        out_shape=jax.ShapeDtypeStruct((B // 2, 2 * E), jnp.float32),
    )(sums128, w2, b2)


_PERM = np.concatenate([np.arange(0, 16), np.arange(32, 48),
                        np.arange(16, 32), np.arange(48, 64)])


def kernel(items, query_words, word_emb, q_weight, q_bias):
    u = lax.bitcast_convert_type(word_emb, jnp.uint32)
    r = (u + jnp.uint32(0x7FFF) + ((u >> 16) & jnp.uint32(1))) >> 16
    pk = r[:, :32] | (r[:, 32:] << 16)
    table_i32 = lax.bitcast_convert_type(pk, jnp.int32)

    qw128 = query_words.reshape(B * H // 128, 128)
    sums128 = _sc_sums(table_i32, qw128)

    wt = (q_weight.T * (1.0 / H))[_PERM, :]
    w2 = jnp.zeros((2 * E, 2 * E), wt.dtype)
    w2 = w2.at[:E, :E].set(wt).at[E:, E:].set(wt)
    b2 = jnp.concatenate([q_bias, q_bias]).reshape(1, 2 * E)
    out128 = _proj(sums128, w2, b2)
    return out128.reshape(B, E)


# speedup vs baseline: 1.1752x; 1.1752x over previous
import jax
import jax.numpy as jnp
from jax import lax
from jax.experimental import pallas as pl
from jax.experimental.pallas import tpu as pltpu
from jax.experimental.pallas import tpu_sc as plsc

B = 16384
H = 200
E = 64
V = 100000
NW = 32
RW = B // NW
C = 16
NCHUNK = RW // C
NIDX = C * H // 128
HC = C // 2
FLAT = HC * H

VP = V // 2
CVT = 256
NFULL = VP // CVT
TAIL = VP - NFULL * CVT
NPW_HI = NFULL - NW * (NFULL // NW)


def _pack_rows(src, nrows, dst, dst_base):
    def r_body(r, carry):
        for half in range(2):
            for g in range(2):
                o = 64 * half + 32 * g
                p = plsc.pack(src[r, pl.ds(o, 16)], src[r, pl.ds(o + 16, 16)],
                              format=plsc.PackFormat.INTERLEAVED)
                dst[dst_base + 2 * r + half, pl.ds(32 * g, 32)] = p
        return carry
    lax.fori_loop(0, nrows, r_body, 0, unroll=4)


def _cvt_body(wp_hbm, wb_hbm, in0, in1, ob0, ob1, s0, s1, so0, so1):
    wid = lax.axis_index("s") * 2 + lax.axis_index("c")
    niter = jnp.where(wid < NPW_HI, NFULL // NW + 1, NFULL // NW)
    in_b, ob_b = (in0, in1), (ob0, ob1)
    s_b, so_b = (s0, s1), (so0, so1)

    def fire_in(k, p):
        pltpu.async_copy(wp_hbm.at[pl.ds((wid + NW * k) * CVT, CVT)],
                         in_b[p], s_b[p])

    def drain_in(p):
        pltpu.make_async_copy(wp_hbm.at[pl.ds(0, CVT)],
                              in_b[p], s_b[p]).wait()

    def fire_out(k, p):
        pltpu.async_copy(ob_b[p],
                         wb_hbm.at[pl.ds((wid + NW * k) * 2 * CVT, 2 * CVT)],
                         so_b[p])

    def drain_out(p):
        pltpu.make_async_copy(ob_b[p], wb_hbm.at[pl.ds(0, 2 * CVT)],
                              so_b[p]).wait()

    fire_in(0, 0)

    def k_body(k, carry):
        def do(p):
            @pl.when(k + 1 < niter)
            def _():
                fire_in(k + 1, 1 - p)
            drain_in(p)

            @pl.when(k >= 2)
            def _():
                drain_out(p)
            _pack_rows(in_b[p], CVT, ob_b[p], 0)
            fire_out(k, p)
        lax.cond(lax.rem(k, 2) == 0, lambda: do(0), lambda: do(1))
        return carry

    lax.fori_loop(0, niter, k_body, 0)
    drain_out(0)
    drain_out(1)

    @pl.when(wid == NW - 1)
    def _():
        pltpu.sync_copy(wp_hbm.at[pl.ds(NFULL * CVT, TAIL)],
                        in0.at[pl.ds(0, TAIL)])
        _pack_rows(in0, TAIL, ob0, 0)
        pltpu.sync_copy(ob0.at[pl.ds(0, 2 * TAIL)],
                        wb_hbm.at[pl.ds(NFULL * 2 * CVT, 2 * TAIL)])


@jax.jit
def _sc_cvt(wp):
    mesh = plsc.VectorSubcoreMesh(core_axis_name="c", subcore_axis_name="s")
    f = pl.kernel(
        _cvt_body,
        mesh=mesh,
        compiler_params=pltpu.CompilerParams(
            use_tc_tiling_on_sc=False, needs_layout_passes=False),
        out_type=jax.ShapeDtypeStruct((V, E), jnp.bfloat16),
        scratch_types=[
            pltpu.VMEM((CVT, 128), jnp.float32),
            pltpu.VMEM((CVT, 128), jnp.float32),
            pltpu.VMEM((2 * CVT, E), jnp.bfloat16),
            pltpu.VMEM((2 * CVT, E), jnp.bfloat16),
            pltpu.SemaphoreType.DMA,
            pltpu.SemaphoreType.DMA,
            pltpu.SemaphoreType.DMA,
            pltpu.SemaphoreType.DMA,
        ],
    )
    return f(wp)


_G0 = [(k, 0, 128, 128 * k) for k in range(12)] + [(12, 0, 64, 1536)]
_G1 = [(12, 64, 64, 0)] + [(13 + k, 0, 128, 64 + 128 * k) for k in range(12)]
_GATHERS = (_G0, _G1)


def _sc_body(table_hbm, qw_hbm, out_hbm,
             idx0, idx1, embA, embB, out_v, semA, semB):
    wid = lax.axis_index("s") * 2 + lax.axis_index("c")
    base = wid * RW
    idx_b = (idx0, idx1)
    emb_b = (embA, embB)
    sem_b = (semA, semB)

    def stage_idx(ci, p):
        pltpu.sync_copy(qw_hbm.at[pl.ds((base + ci * C) * H // 128, NIDX)],
                        idx_b[p])

    def fire(half, p, eb):
        for row, coff, n, doff in _GATHERS[half]:
            src = (idx_b[p].at[row] if n == 128
                   else idx_b[p].at[row, pl.ds(coff, n)])
            pltpu.async_copy(table_hbm.at[src],
                             emb_b[eb].at[pl.ds(doff, n)], sem_b[eb])

    def drain(half, p, eb):
        for row, coff, n, doff in _GATHERS[half]:
            src = (idx_b[p].at[row] if n == 128
                   else idx_b[p].at[row, pl.ds(coff, n)])
            pltpu.make_async_copy(table_hbm.at[src],
                                  emb_b[eb].at[pl.ds(doff, n)],
                                  sem_b[eb]).wait()

    def accum(half, eb):
        emb_v = emb_b[eb]
        for q in range(HC):
            r = half * HC + q

            def j_body(j, accs):
                e0, o0, e1, o1 = accs
                off = q * H + 4 * j
                for g, sel in ((0, 0), (32, 1)):
                    s = ((emb_v[off, pl.ds(g, 32)] +
                          emb_v[off + 1, pl.ds(g, 32)]) +
                         (emb_v[off + 2, pl.ds(g, 32)] +
                          emb_v[off + 3, pl.ds(g, 32)]))
                    a, bb = plsc.unpack(s, format=plsc.PackFormat.INTERLEAVED)
                    if sel == 0:
                        e0, o0 = e0 + a, o0 + bb
                    else:
                        e1, o1 = e1 + a, o1 + bb
                return (e0, o0, e1, o1)

            z = jnp.zeros((16,), jnp.float32)
            e0, o0, e1, o1 = lax.fori_loop(0, H // 4, j_body, (z, z, z, z),
                                           unroll=5)
            cb = 64 * (r % 2)
            out_v[r // 2, pl.ds(cb, 16)] = e0
            out_v[r // 2, pl.ds(cb + 16, 16)] = o0
            out_v[r // 2, pl.ds(cb + 32, 16)] = e1
            out_v[r // 2, pl.ds(cb + 48, 16)] = o1

    def flush_out(ci):
        pltpu.sync_copy(out_v, out_hbm.at[pl.ds((base + ci * C) // 2, HC)])

    stage_idx(0, 0)
    fire(0, 0, 0)

    def chunk_body(ci, carry):
        def do(p):
            fire(1, p, 1)
            stage_idx(ci + 1, 1 - p)
            drain(0, p, 0)
            accum(0, 0)
            fire(0, 1 - p, 0)
            drain(1, p, 1)
            accum(1, 1)
            flush_out(ci)
        lax.cond(lax.rem(ci, 2) == 0, lambda: do(0), lambda: do(1))
        return carry

    lax.fori_loop(0, NCHUNK - 1, chunk_body, 0)
    pl_last = (NCHUNK - 1) % 2
    fire(1, pl_last, 1)
    drain(0, pl_last, 0)
    accum(0, 0)
    drain(1, pl_last, 1)
    accum(1, 1)
    flush_out(NCHUNK - 1)


@jax.jit
def _sc_sums(table_bf16, qw128):
    mesh = plsc.VectorSubcoreMesh(core_axis_name="c", subcore_axis_name="s")
    f = pl.kernel(
        _sc_body,
        mesh=mesh,
        compiler_params=pltpu.CompilerParams(
            use_tc_tiling_on_sc=False, needs_layout_passes=False),
        out_type=jax.ShapeDtypeStruct((B // 2, 2 * E), jnp.float32),
        scratch_types=[
            pltpu.VMEM((NIDX, 128), jnp.int32),
            pltpu.VMEM((NIDX, 128), jnp.int32),
            pltpu.VMEM((FLAT, E), jnp.bfloat16),
            pltpu.VMEM((FLAT, E), jnp.bfloat16),
            pltpu.VMEM((HC, 2 * E), jnp.float32),
            pltpu.SemaphoreType.DMA,
            pltpu.SemaphoreType.DMA,
        ],
    )
    return f(table_bf16, qw128)


def _proj_body(x_ref, w_ref, b_ref, o_ref):
    o_ref[...] = jnp.tanh(
        jnp.dot(x_ref[...], w_ref[...], preferred_element_type=jnp.float32)
        + b_ref[...])


@jax.jit
def _proj(sums128, w2, b2):
    blk = 2048
    return pl.pallas_call(
        _proj_body,
        grid=(B // 2 // blk,),
        in_specs=[
            pl.BlockSpec((blk, 2 * E), lambda i: (i, 0)),
            pl.BlockSpec((2 * E, 2 * E), lambda i: (0, 0)),
            pl.BlockSpec((1, 2 * E), lambda i: (0, 0)),
        ],
        out_specs=pl.BlockSpec((blk, 2 * E), lambda i: (i, 0)),
        out_shape=jax.ShapeDtypeStruct((B // 2, 2 * E), jnp.float32),
    )(sums128, w2, b2)


def kernel(items, query_words, word_emb, q_weight, q_bias):
    wb = _sc_cvt(word_emb.reshape(V // 2, 2 * E))
    qw128 = query_words.reshape(B * H // 128, 128)
    sums128 = _sc_sums(wb, qw128)
    wt = q_weight.T * (1.0 / H)
    w2 = jnp.zeros((2 * E, 2 * E), wt.dtype)
    w2 = w2.at[:E, :E].set(wt).at[E:, E:].set(wt)
    b2 = jnp.concatenate([q_bias, q_bias]).reshape(1, 2 * E)
    out128 = _proj(sums128, w2, b2)
    return out128.reshape(B, E)


# speedup vs baseline: 1.1907x; 1.0132x over previous
import numpy as np

import jax
import jax.numpy as jnp
from jax import lax
from jax.experimental import pallas as pl
from jax.experimental.pallas import tpu as pltpu
from jax.experimental.pallas import tpu_sc as plsc

B = 16384
H = 200
E = 64
NW = 32
RW = B // NW
C = 16
NCHUNK = RW // C
NIDX = C * H // 128
HC = C // 2
FLAT = HC * H

_G0 = [(k, 0, 128, 128 * k) for k in range(12)] + [(12, 0, 64, 1536)]
_G1 = [(12, 64, 64, 0)] + [(13 + k, 0, 128, 64 + 128 * k) for k in range(12)]
_GATHERS = (_G0, _G1)


def _sc_body(table_hbm, qw_hbm, out_hbm,
             idx0, idx1, embA, embB, out_v, semA, semB):
    wid = lax.axis_index("s") * 2 + lax.axis_index("c")
    base = wid * RW
    idx_b = (idx0, idx1)
    emb_b = (embA, embB)
    sem_b = (semA, semB)

    def stage_idx(ci, p):
        pltpu.sync_copy(qw_hbm.at[pl.ds((base + ci * C) * H // 128, NIDX)],
                        idx_b[p])

    def fire(half, p, eb):
        for row, coff, n, doff in _GATHERS[half]:
            src = (idx_b[p].at[row] if n == 128
                   else idx_b[p].at[row, pl.ds(coff, n)])
            pltpu.async_copy(table_hbm.at[src],
                             emb_b[eb].at[pl.ds(doff, n)], sem_b[eb])

    def drain(half, p, eb):
        for row, coff, n, doff in _GATHERS[half]:
            src = (idx_b[p].at[row] if n == 128
                   else idx_b[p].at[row, pl.ds(coff, n)])
            pltpu.make_async_copy(table_hbm.at[src],
                                  emb_b[eb].at[pl.ds(doff, n)],
                                  sem_b[eb]).wait()

    def accum(half, eb):
        emb_v = emb_b[eb]
        for q in range(HC):
            r = half * HC + q

            def j_body(j, accs):
                e0, o0, e1, o1 = accs
                off = q * H + 4 * j
                for g, sel in ((0, 0), (32, 1)):
                    s = ((emb_v[off, pl.ds(g, 32)] +
                          emb_v[off + 1, pl.ds(g, 32)]) +
                         (emb_v[off + 2, pl.ds(g, 32)] +
                          emb_v[off + 3, pl.ds(g, 32)]))
                    a, bb = plsc.unpack(s, format=plsc.PackFormat.INTERLEAVED)
                    if sel == 0:
                        e0, o0 = e0 + a, o0 + bb
                    else:
                        e1, o1 = e1 + a, o1 + bb
                return (e0, o0, e1, o1)

            z = jnp.zeros((16,), jnp.float32)
            e0, o0, e1, o1 = lax.fori_loop(0, H // 4, j_body, (z, z, z, z),
                                           unroll=5)
            cb = 64 * (r % 2)
            out_v[r // 2, pl.ds(cb, 16)] = e0
            out_v[r // 2, pl.ds(cb + 16, 16)] = o0
            out_v[r // 2, pl.ds(cb + 32, 16)] = e1
            out_v[r // 2, pl.ds(cb + 48, 16)] = o1

    def flush_out(ci):
        pltpu.sync_copy(out_v, out_hbm.at[pl.ds((base + ci * C) // 2, HC)])

    stage_idx(0, 0)
    fire(0, 0, 0)

    def chunk_body(ci, carry):
        p = lax.rem(ci, 2)

        def do(p):
            fire(1, p, 1)
            stage_idx(ci + 1, 1 - p)
            drain(0, p, 0)
            accum(0, 0)
            fire(0, 1 - p, 0)
            drain(1, p, 1)
            accum(1, 1)
            flush_out(ci)

        lax.cond(p == 0, lambda: do(0), lambda: do(1))
        return carry

    lax.fori_loop(0, NCHUNK - 1, chunk_body, 0)
    pl_last = (NCHUNK - 1) % 2
    fire(1, pl_last, 1)
    drain(0, pl_last, 0)
    accum(0, 0)
    drain(1, pl_last, 1)
    accum(1, 1)
    flush_out(NCHUNK - 1)


@jax.jit
def _sc_sums(table_bf16, qw128):
    mesh = plsc.VectorSubcoreMesh(core_axis_name="c", subcore_axis_name="s")
    f = pl.kernel(
        _sc_body,
        mesh=mesh,
        compiler_params=pltpu.CompilerParams(
            use_tc_tiling_on_sc=False, needs_layout_passes=False),
        out_type=jax.ShapeDtypeStruct((B // 2, 2 * E), jnp.float32),
        scratch_types=[
            pltpu.VMEM((NIDX, 128), jnp.int32),
            pltpu.VMEM((NIDX, 128), jnp.int32),
            pltpu.VMEM((FLAT, E), jnp.bfloat16),
            pltpu.VMEM((FLAT, E), jnp.bfloat16),
            pltpu.VMEM((HC, 2 * E), jnp.float32),
            pltpu.SemaphoreType.DMA,
            pltpu.SemaphoreType.DMA,
        ],
    )
    return f(table_bf16, qw128)


def _proj_body(x_ref, w_ref, b_ref, o_ref):
    o_ref[...] = jnp.tanh(
        jnp.dot(x_ref[...], w_ref[...], preferred_element_type=jnp.float32)
        + b_ref[...])


@jax.jit
def _proj(sums128, w2, b2):
    blk = 2048
    return pl.pallas_call(
        _proj_body,
        grid=(B // 2 // blk,),
        in_specs=[
            pl.BlockSpec((blk, 2 * E), lambda i: (i, 0)),
            pl.BlockSpec((2 * E, 2 * E), lambda i: (0, 0)),
            pl.BlockSpec((1, 2 * E), lambda i: (0, 0)),
        ],
        out_specs=pl.BlockSpec((blk, 2 * E), lambda i: (i, 0)),
        out_shape=jax.ShapeDtypeStruct((B // 2, 2 * E), jnp.float32),
    )(sums128, w2, b2)


_PERM = np.concatenate([np.arange(0, 32, 2), np.arange(1, 32, 2),
                        np.arange(32, 64, 2), np.arange(33, 64, 2)])


def kernel(items, query_words, word_emb, q_weight, q_bias):
    qw128 = query_words.reshape(B * H // 128, 128)
    sums128 = _sc_sums(word_emb.astype(jnp.bfloat16), qw128)
    wt = (q_weight.T * (1.0 / H))[_PERM, :]
    w2 = jnp.zeros((2 * E, 2 * E), wt.dtype)
    w2 = w2.at[:E, :E].set(wt).at[E:, E:].set(wt)
    b2 = jnp.concatenate([q_bias, q_bias]).reshape(1, 2 * E)
    out128 = _proj(sums128, w2, b2)
    return out128.reshape(B, E)
